# baseline (speedup 1.0000x reference)
import functools

import jax
import jax.numpy as jnp
from jax import lax
from jax.experimental import pallas as pl
from jax.experimental.pallas import tpu as pltpu
from jax.experimental.pallas import tpu_sc as plsc

N_FIELDS = 26
VOCAB = 100000
EMBED_DIM = 16
N_DENSE = 13
BATCH = 16384

VTILE = 512
VBLOCKS = (VOCAB + VTILE - 1) // VTILE
VPAD = VBLOCKS * VTILE

NUM_CORES = 2
NUM_SUBCORES = 16
NUM_WORKERS = NUM_CORES * NUM_SUBCORES
ROWS_PER_WORKER = BATCH // NUM_WORKERS
IDX_PER_WORKER = N_FIELDS * ROWS_PER_WORKER
ND_PAD = 14


def _planes_body(tt_ref, w_ref, a_ref, d_ref):
    t = tt_ref[0]
    w = w_ref[pl.program_id(0), :]
    sl = pl.ds(0, VOCAB)
    ones = jnp.ones((16,), jnp.float32)
    zeros = jnp.zeros((16,), jnp.float32)
    w1 = jnp.stack([ones, w])
    w2 = jnp.stack([zeros, -0.5 * ones])
    dn = (((1,), (0,)), ((), ()))
    res = (
        jax.lax.dot_general(w1, t, dn, preferred_element_type=jnp.float32)
        + jax.lax.dot_general(w2, t * t, dn, preferred_element_type=jnp.float32)
    )
    a_ref[sl] = res[0]
    d_ref[sl] = res[1]


def _fm_body(idx_hbm, dense_hbm, a_hbm, d_hbm, wv_hbm, out_hbm,
             idx_v, av, ddv, dv, wvb, out_v, sem):
    wid = lax.axis_index("s") * NUM_CORES + lax.axis_index("c")
    pltpu.sync_copy(wv_hbm, wvb)
    w_vecs = [wvb[pl.ds(k * 16, 16)] for k in range(ND_PAD)]

    base = wid * ROWS_PER_WORKER
    pltpu.sync_copy(idx_hbm.at[pl.ds(wid * IDX_PER_WORKER, IDX_PER_WORKER)],
                    idx_v)
    copies = [pltpu.async_copy(
        dense_hbm.at[pl.ds(wid * (ND_PAD * ROWS_PER_WORKER),
                           ND_PAD * ROWS_PER_WORKER)], dv, sem)]
    for j in range(IDX_PER_WORKER // 128):
        isl = idx_v.at[pl.ds(j * 128, 128)]
        dsl = pl.ds(j * 128, 128)
        copies.append(pltpu.async_copy(a_hbm.at[isl], av.at[dsl], sem))
        copies.append(pltpu.async_copy(d_hbm.at[isl], ddv.at[dsl], sem))
    for cp in copies:
        cp.wait()

    for g in range(ROWS_PER_WORKER // 16):
        g0 = g * 16
        acc_a = jnp.zeros((16,), jnp.float32)
        acc_d = jnp.zeros((16,), jnp.float32)
        for f in range(N_FIELDS):
            off = pl.ds(f * ROWS_PER_WORKER + g0, 16)
            acc_a = acc_a + av[off]
            acc_d = acc_d + ddv[off]
        dlin = jnp.zeros((16,), jnp.float32)
        for k in range(ND_PAD):
            dlin = dlin + dv[pl.ds(k * ROWS_PER_WORKER + g0, 16)] * w_vecs[k]
        x = acc_d + dlin + 0.5 * acc_a * acc_a
        out_v[pl.ds(g0, 16)] = 1.0 / (1.0 + jnp.exp(-x))

    pltpu.sync_copy(out_v, out_hbm.at[pl.ds(base, ROWS_PER_WORKER)])


@jax.jit
def _fm_call(tables_t, w_mat, idx_flat, dense_flat, wv_flat):
    planes = pl.pallas_call(
        _planes_body,
        grid=(N_FIELDS,),
        in_specs=[
            pl.BlockSpec((1, EMBED_DIM, VOCAB), lambda f: (f, 0, 0)),
            pl.BlockSpec((N_FIELDS, EMBED_DIM), lambda f: (0, 0)),
        ],
        out_specs=[
            pl.BlockSpec((VPAD,), lambda f: (f,)),
            pl.BlockSpec((VPAD,), lambda f: (f,)),
        ],
        out_shape=[
            jax.ShapeDtypeStruct((N_FIELDS * VPAD,), jnp.float32),
            jax.ShapeDtypeStruct((N_FIELDS * VPAD,), jnp.float32),
        ],
    )(tables_t, w_mat)
    a_plane, d_plane = planes

    mesh = plsc.VectorSubcoreMesh(core_axis_name="c", subcore_axis_name="s")
    run = functools.partial(
        pl.kernel,
        out_type=jax.ShapeDtypeStruct((BATCH,), jnp.float32),
        mesh=mesh,
        compiler_params=pltpu.CompilerParams(use_tc_tiling_on_sc=False),
        scratch_types=[
            pltpu.VMEM((IDX_PER_WORKER,), jnp.int32),
            pltpu.VMEM((IDX_PER_WORKER,), jnp.float32),
            pltpu.VMEM((IDX_PER_WORKER,), jnp.float32),
            pltpu.VMEM((ND_PAD * ROWS_PER_WORKER,), jnp.float32),
            pltpu.VMEM((ND_PAD * 16,), jnp.float32),
            pltpu.VMEM((ROWS_PER_WORKER,), jnp.float32),
            pltpu.SemaphoreType.DMA,
        ],
    )(_fm_body)
    return run(idx_flat, dense_flat, a_plane, d_plane, wv_flat)


def kernel(dense_input, sparse_input, tables, w_dense, w_sparse, b):
    tables_t = jnp.transpose(tables, (0, 2, 1))
    w_mat = w_sparse.reshape(N_FIELDS, EMBED_DIM)

    offs = jnp.arange(N_FIELDS, dtype=jnp.int32) * VPAD
    gidx = sparse_input + offs[None, :]
    idx_flat = (
        gidx.reshape(NUM_WORKERS, ROWS_PER_WORKER, N_FIELDS)
        .transpose(0, 2, 1)
        .reshape(-1)
    )

    dense_t = jnp.zeros((ND_PAD, BATCH), jnp.float32)
    dense_t = dense_t.at[:N_DENSE].set(dense_input.T)
    dense_t = dense_t.at[N_DENSE].set(1.0)
    dense_flat = (
        dense_t.reshape(ND_PAD, NUM_WORKERS, ROWS_PER_WORKER)
        .transpose(1, 0, 2)
        .reshape(-1)
    )
    wd = jnp.zeros((ND_PAD,), jnp.float32)
    wd = wd.at[:N_DENSE].set(w_dense[:, 0])
    wd = wd.at[N_DENSE].set(b[0])
    wv_flat = jnp.repeat(wd, 16)

    out = _fm_call(tables_t, w_mat, idx_flat, dense_flat, wv_flat)
    return out.reshape(BATCH, 1)


# speedup vs baseline: 1.0285x; 1.0285x over previous
import functools

import jax
import jax.numpy as jnp
from jax import lax
from jax.experimental import pallas as pl
from jax.experimental.pallas import tpu as pltpu
from jax.experimental.pallas import tpu_sc as plsc

N_FIELDS = 26
VOCAB = 100000
EMBED_DIM = 16
N_DENSE = 13
BATCH = 16384

VTILE = 512
VBLOCKS = (VOCAB + VTILE - 1) // VTILE
VPAD = VBLOCKS * VTILE

NUM_CORES = 2
NUM_SUBCORES = 16
NUM_WORKERS = NUM_CORES * NUM_SUBCORES
ROWS_PER_WORKER = BATCH // NUM_WORKERS
IDX_PER_WORKER = N_FIELDS * ROWS_PER_WORKER
ND_PAD = 14


def _planes_body(tt_ref, w_ref, a_ref, d_ref):
    t = tt_ref[0]
    w = w_ref[pl.program_id(0), :]
    sl = pl.ds(0, VOCAB)
    ones = jnp.ones((16,), jnp.float32)
    zeros = jnp.zeros((16,), jnp.float32)
    w1 = jnp.stack([ones, w])
    w2 = jnp.stack([zeros, -0.5 * ones])
    dn = (((1,), (0,)), ((), ()))
    res = (
        jax.lax.dot_general(w1, t, dn, preferred_element_type=jnp.float32)
        + jax.lax.dot_general(w2, t * t, dn, preferred_element_type=jnp.float32)
    )
    a_ref[sl] = res[0]
    d_ref[sl] = res[1]


CHUNK = 128
NUM_CHUNKS = ROWS_PER_WORKER // CHUNK
IDX_PER_CHUNK = N_FIELDS * CHUNK


def _fm_body(idx_hbm, dense_hbm, a_hbm, d_hbm, wv_hbm, out_hbm,
             idx_v, av, ddv, dv, wvb, out_v, sem0, sem1):
    sems = [sem0, sem1]
    wid = lax.axis_index("s") * NUM_CORES + lax.axis_index("c")
    pltpu.sync_copy(wv_hbm, wvb)
    w_vecs = [wvb[pl.ds(k * 16, 16)] for k in range(ND_PAD)]

    base = wid * ROWS_PER_WORKER
    pltpu.sync_copy(idx_hbm.at[pl.ds(wid * IDX_PER_WORKER, IDX_PER_WORKER)],
                    idx_v)
    dcopy = pltpu.async_copy(
        dense_hbm.at[pl.ds(wid * (ND_PAD * ROWS_PER_WORKER),
                           ND_PAD * ROWS_PER_WORKER)], dv, sem0)

    def fire(c):
        b = c % 2
        hs = []
        for j in range(IDX_PER_CHUNK // 128):
            isl = idx_v.at[pl.ds(c * IDX_PER_CHUNK + j * 128, 128)]
            dsl = pl.ds(b * IDX_PER_CHUNK + j * 128, 128)
            hs.append(pltpu.async_copy(a_hbm.at[isl], av.at[dsl], sems[b]))
            hs.append(pltpu.async_copy(d_hbm.at[isl], ddv.at[dsl], sems[b]))
        return hs

    handles = fire(0)
    dcopy.wait()
    for c in range(NUM_CHUNKS):
        nxt = fire(c + 1) if c + 1 < NUM_CHUNKS else []
        for cp in handles:
            cp.wait()
        handles = nxt
        b0 = (c % 2) * IDX_PER_CHUNK
        for g in range(CHUNK // 16):
            g0 = g * 16
            acc_a = jnp.zeros((16,), jnp.float32)
            acc_d = jnp.zeros((16,), jnp.float32)
            for f in range(N_FIELDS):
                off = pl.ds(b0 + f * CHUNK + g0, 16)
                acc_a = acc_a + av[off]
                acc_d = acc_d + ddv[off]
            dlin = jnp.zeros((16,), jnp.float32)
            for k in range(ND_PAD):
                dlin = dlin + (
                    dv[pl.ds(k * ROWS_PER_WORKER + c * CHUNK + g0, 16)]
                    * w_vecs[k]
                )
            x = acc_d + dlin + 0.5 * acc_a * acc_a
            out_v[pl.ds(c * CHUNK + g0, 16)] = 1.0 / (1.0 + jnp.exp(-x))

    pltpu.sync_copy(out_v, out_hbm.at[pl.ds(base, ROWS_PER_WORKER)])


@jax.jit
def _fm_call(tables_t, w_mat, idx_flat, dense_flat, wv_flat):
    planes = pl.pallas_call(
        _planes_body,
        grid=(N_FIELDS,),
        in_specs=[
            pl.BlockSpec((1, EMBED_DIM, VOCAB), lambda f: (f, 0, 0)),
            pl.BlockSpec((N_FIELDS, EMBED_DIM), lambda f: (0, 0)),
        ],
        out_specs=[
            pl.BlockSpec((VPAD,), lambda f: (f,)),
            pl.BlockSpec((VPAD,), lambda f: (f,)),
        ],
        out_shape=[
            jax.ShapeDtypeStruct((N_FIELDS * VPAD,), jnp.float32),
            jax.ShapeDtypeStruct((N_FIELDS * VPAD,), jnp.float32),
        ],
    )(tables_t, w_mat)
    a_plane, d_plane = planes

    mesh = plsc.VectorSubcoreMesh(core_axis_name="c", subcore_axis_name="s")
    run = functools.partial(
        pl.kernel,
        out_type=jax.ShapeDtypeStruct((BATCH,), jnp.float32),
        mesh=mesh,
        compiler_params=pltpu.CompilerParams(use_tc_tiling_on_sc=False),
        scratch_types=[
            pltpu.VMEM((IDX_PER_WORKER,), jnp.int32),
            pltpu.VMEM((2 * IDX_PER_CHUNK,), jnp.float32),
            pltpu.VMEM((2 * IDX_PER_CHUNK,), jnp.float32),
            pltpu.VMEM((ND_PAD * ROWS_PER_WORKER,), jnp.float32),
            pltpu.VMEM((ND_PAD * 16,), jnp.float32),
            pltpu.VMEM((ROWS_PER_WORKER,), jnp.float32),
            pltpu.SemaphoreType.DMA,
            pltpu.SemaphoreType.DMA,
        ],
    )(_fm_body)
    return run(idx_flat, dense_flat, a_plane, d_plane, wv_flat)


def kernel(dense_input, sparse_input, tables, w_dense, w_sparse, b):
    tables_t = jnp.transpose(tables, (0, 2, 1))
    w_mat = w_sparse.reshape(N_FIELDS, EMBED_DIM)

    offs = jnp.arange(N_FIELDS, dtype=jnp.int32) * VPAD
    gidx = sparse_input + offs[None, :]
    idx_flat = (
        gidx.reshape(NUM_WORKERS, ROWS_PER_WORKER, N_FIELDS)
        .transpose(0, 2, 1)
        .reshape(-1)
    )

    dense_t = jnp.zeros((ND_PAD, BATCH), jnp.float32)
    dense_t = dense_t.at[:N_DENSE].set(dense_input.T)
    dense_t = dense_t.at[N_DENSE].set(1.0)
    dense_flat = (
        dense_t.reshape(ND_PAD, NUM_WORKERS, ROWS_PER_WORKER)
        .transpose(1, 0, 2)
        .reshape(-1)
    )
    wd = jnp.zeros((ND_PAD,), jnp.float32)
    wd = wd.at[:N_DENSE].set(w_dense[:, 0])
    wd = wd.at[N_DENSE].set(b[0])
    wv_flat = jnp.repeat(wd, 16)

    out = _fm_call(tables_t, w_mat, idx_flat, dense_flat, wv_flat)
    return out.reshape(BATCH, 1)
